# Initial kernel scaffold; baseline (speedup 1.0000x reference)
import jax
import jax.numpy as jnp
from jax.experimental import pallas as pl


def kernel(features, edge_index, Wself0, Wneigh0, b0, Wself1, Wneigh1, b1, Wself2, Wneigh2, b2):
    raise NotImplementedError("write your pallas kernel here")


# speedup vs baseline: 6.8879x; 6.8879x over previous
import jax
import jax.numpy as jnp
from jax import lax
from jax.experimental import pallas as pl
from jax.experimental.pallas import tpu as pltpu
from jax.experimental.pallas import tpu_sc as plsc

N = 10000
E = 320000
D = 128
H = 128
C = 47

NC = 2
NS = 16
NWORK = NC * NS
EPW = E // NWORK
CH = 80
NCHUNK = EPW // CH
NBUF = 4
NGRP = (NCHUNK - 1) // NBUF
PUB = 1000
NPUB = N // PUB
ZR = 40
NZ = PUB // ZR


def _sc_agg_body(y_hbm, edges_hbm, out_hbm, idxb, rows, zrows, semi, semg,
                 sems, agg_sh):
    cid = lax.axis_index("c")
    sid = lax.axis_index("s")
    wid = cid * NS + sid

    def _zero_row(i, _):
        for j in range(H // 16):
            zrows[i, pl.ds(j * 16, 16)] = jnp.zeros((16,), jnp.float32)
        return 0
    lax.fori_loop(0, ZR, _zero_row, 0)
    r0 = sid * PUB

    @pl.when(sid < NPUB)
    def _():
        for k in range(NZ):
            pltpu.sync_copy(zrows, agg_sh.at[pl.ds(r0 + k * ZR, ZR), :])
    plsc.subcore_barrier()

    def _start_idx(b, c):
        pltpu.async_copy(edges_hbm.at[wid, c], idxb.at[b], semi.at[b])

    def _wait_idx(b):
        pltpu.make_async_copy(edges_hbm.at[wid, 0], idxb.at[b],
                              semi.at[b]).wait()

    def _start_gather(b):
        pltpu.async_copy(y_hbm.at[idxb.at[b, 0]], rows.at[b], semg.at[b])

    def _wait_gather(b):
        pltpu.make_async_copy(y_hbm.at[idxb.at[b, 0]], rows.at[b],
                              semg.at[b]).wait()

    def _start_scatter(b):
        pltpu.async_copy(rows.at[b], agg_sh.at[idxb.at[b, 1]], sems.at[b],
                         add=True)

    def _wait_scatter(b):
        pltpu.make_async_copy(rows.at[b], agg_sh.at[idxb.at[b, 1]],
                              sems.at[b]).wait()

    for b in range(NBUF):
        _start_idx(b, b)
    for b in range(NBUF):
        _wait_idx(b)
        _start_gather(b)
    for b in range(NBUF):
        _wait_gather(b)
        _start_idx(b, NBUF + b)
        _start_scatter(b)

    def _group(g, _):
        c0 = g * NBUF
        for b in range(NBUF):
            _wait_idx(b)
            _wait_scatter(b)
            _start_gather(b)
        for b in range(NBUF):
            _wait_gather(b)

            @pl.when(c0 + NBUF + b < NCHUNK)
            def _():
                _start_idx(b, c0 + NBUF + b)
            _start_scatter(b)
        return 0
    lax.fori_loop(1, NGRP, _group, 0)

    _wait_idx(0)
    _wait_scatter(0)
    _start_gather(0)
    _wait_gather(0)
    _start_scatter(0)
    _wait_scatter(0)
    for b in range(1, NBUF):
        _wait_scatter(b)
    plsc.subcore_barrier()

    @pl.when(sid < NPUB)
    def _():
        sl = pl.ds(r0, PUB)
        pltpu.sync_copy(agg_sh.at[sl, :], out_hbm.at[cid, sl, :])


def _sc_agg_sync_body(y_hbm, src_hbm, dst_hbm, out_hbm, srcv, dstv, rows,
                      zrows, sem, agg_sh):
    cid = lax.axis_index("c")
    sid = lax.axis_index("s")
    wid = cid * NS + sid

    def _zero_row(i, _):
        for j in range(H // 16):
            zrows[i, pl.ds(j * 16, 16)] = jnp.zeros((16,), jnp.float32)
        return 0
    lax.fori_loop(0, ZR, _zero_row, 0)
    r0 = sid * PUB

    @pl.when(sid < NPUB)
    def _():
        for k in range(NZ):
            pltpu.sync_copy(zrows, agg_sh.at[pl.ds(r0 + k * ZR, ZR), :])
    plsc.subcore_barrier()

    pltpu.sync_copy(src_hbm.at[wid], srcv)
    pltpu.sync_copy(dst_hbm.at[wid], dstv)

    def _chunk(j, _):
        pltpu.async_copy(y_hbm.at[srcv.at[j]], rows, sem).wait()
        pltpu.sync_copy(rows, agg_sh.at[dstv.at[j]], add=True)
        return 0
    lax.fori_loop(0, NCHUNK, _chunk, 0)
    plsc.subcore_barrier()

    @pl.when(sid < NPUB)
    def _():
        sl = pl.ds(r0, PUB)
        pltpu.sync_copy(agg_sh.at[sl, :], out_hbm.at[cid, sl, :])


_sc_agg_sync = pl.kernel(
    _sc_agg_sync_body,
    out_type=jax.ShapeDtypeStruct((NC, N, H), jnp.float32),
    mesh=plsc.VectorSubcoreMesh(core_axis_name="c", subcore_axis_name="s"),
    scratch_types=[
        pltpu.VMEM((NCHUNK, CH), jnp.int32),
        pltpu.VMEM((NCHUNK, CH), jnp.int32),
        pltpu.VMEM((CH, H), jnp.float32),
        pltpu.VMEM((ZR, H), jnp.float32),
        pltpu.SemaphoreType.DMA,
        pltpu.VMEM_SHARED((N, H), jnp.float32),
    ],
)


_sc_agg = pl.kernel(
    _sc_agg_body,
    out_type=jax.ShapeDtypeStruct((NC, N, H), jnp.float32),
    mesh=plsc.VectorSubcoreMesh(core_axis_name="c", subcore_axis_name="s"),
    scratch_types=[
        pltpu.VMEM((NBUF, 2, CH), jnp.int32),
        pltpu.VMEM((NBUF, CH, H), jnp.float32),
        pltpu.VMEM((ZR, H), jnp.float32),
        pltpu.SemaphoreType.DMA((NBUF,)),
        pltpu.SemaphoreType.DMA((NBUF,)),
        pltpu.SemaphoreType.DMA((NBUF,)),
        pltpu.VMEM_SHARED((N, H), jnp.float32),
    ],
)


def _sc_deg_body(dst_hbm, deg_hbm, dstv, hist):
    cid = lax.axis_index("c")
    sid = lax.axis_index("s")
    wid = cid * NS + sid

    def _zero(i, _):
        hist[pl.ds(i * 16, 16)] = jnp.zeros((16,), jnp.float32)
        return 0
    lax.fori_loop(0, N // 16, _zero, 0)

    pltpu.sync_copy(dst_hbm.at[wid], dstv)
    ones = jnp.ones((16,), jnp.float32)

    def _edges(i, _):
        def _vec(k, _):
            idx = dstv[i, pl.ds(k * 16, 16)]
            plsc.addupdate_scatter(hist, [idx], ones)
            return 0
        lax.fori_loop(0, CH // 16, _vec, 0)
        return 0
    lax.fori_loop(0, NCHUNK, _edges, 0)

    pltpu.sync_copy(hist, deg_hbm.at[wid])


_sc_deg = pl.kernel(
    _sc_deg_body,
    out_type=jax.ShapeDtypeStruct((NWORK, N), jnp.float32),
    mesh=plsc.VectorSubcoreMesh(core_axis_name="c", subcore_axis_name="s"),
    scratch_types=[
        pltpu.VMEM((NCHUNK, CH), jnp.int32),
        pltpu.VMEM((N,), jnp.float32),
    ],
    compiler_params=pltpu.CompilerParams(needs_layout_passes=False),
)


_BLK = 1000
_GRID = N // _BLK


def _mm0_body(x_ref, wn_ref, ws_ref, y_ref, s_ref):
    x = x_ref[...]
    y_ref[...] = jnp.dot(x, wn_ref[...].T, preferred_element_type=jnp.float32)
    s_ref[...] = jnp.dot(x, ws_ref[...].T, preferred_element_type=jnp.float32)


def _mm0(x, wn, ws):
    return pl.pallas_call(
        _mm0_body,
        grid=(_GRID,),
        in_specs=[
            pl.BlockSpec((_BLK, D), lambda i: (i, 0)),
            pl.BlockSpec((H, D), lambda i: (0, 0)),
            pl.BlockSpec((H, D), lambda i: (0, 0)),
        ],
        out_specs=[
            pl.BlockSpec((_BLK, H), lambda i: (i, 0)),
            pl.BlockSpec((_BLK, H), lambda i: (i, 0)),
        ],
        out_shape=[jax.ShapeDtypeStruct((N, H), jnp.float32),
                   jax.ShapeDtypeStruct((N, H), jnp.float32)],
    )(x, wn, ws)


def _mid_body(s_ref, agg_ref, deg_ref, b_ref, wn_ref, ws_ref, y_ref, sn_ref):
    agg = agg_ref[0] + agg_ref[1]
    deg = jnp.sum(deg_ref[...], axis=1, keepdims=True)
    inv = 1.0 / jnp.maximum(deg, 1.0)
    h = jnp.maximum(s_ref[...] + agg * inv + b_ref[...], 0.0)
    y_ref[...] = jnp.dot(h, wn_ref[...].T, preferred_element_type=jnp.float32)
    sn_ref[...] = jnp.dot(h, ws_ref[...].T, preferred_element_type=jnp.float32)


def _mid(s, aggp, degp, b, wn, ws):
    return pl.pallas_call(
        _mid_body,
        grid=(_GRID,),
        in_specs=[
            pl.BlockSpec((_BLK, H), lambda i: (i, 0)),
            pl.BlockSpec((NC, _BLK, H), lambda i: (0, i, 0)),
            pl.BlockSpec((_BLK, NWORK), lambda i: (i, 0)),
            pl.BlockSpec((1, H), lambda i: (0, 0)),
            pl.BlockSpec((H, H), lambda i: (0, 0)),
            pl.BlockSpec((H, H), lambda i: (0, 0)),
        ],
        out_specs=[
            pl.BlockSpec((_BLK, H), lambda i: (i, 0)),
            pl.BlockSpec((_BLK, H), lambda i: (i, 0)),
        ],
        out_shape=[jax.ShapeDtypeStruct((N, H), jnp.float32),
                   jax.ShapeDtypeStruct((N, H), jnp.float32)],
    )(s, aggp, degp, b, wn, ws)


def _tail_body(s_ref, agg_ref, deg_ref, b_ref, o_ref):
    agg = agg_ref[0] + agg_ref[1]
    deg = jnp.sum(deg_ref[...], axis=1, keepdims=True)
    inv = 1.0 / jnp.maximum(deg, 1.0)
    o_ref[...] = s_ref[...] + agg * inv + b_ref[...]


def _tail(s, aggp, degp, b):
    return pl.pallas_call(
        _tail_body,
        grid=(_GRID,),
        in_specs=[
            pl.BlockSpec((_BLK, H), lambda i: (i, 0)),
            pl.BlockSpec((NC, _BLK, H), lambda i: (0, i, 0)),
            pl.BlockSpec((_BLK, NWORK), lambda i: (i, 0)),
            pl.BlockSpec((1, H), lambda i: (0, 0)),
        ],
        out_specs=pl.BlockSpec((_BLK, H), lambda i: (i, 0)),
        out_shape=jax.ShapeDtypeStruct((N, H), jnp.float32),
    )(s, aggp, degp, b)


def kernel(features, edge_index, Wself0, Wneigh0, b0, Wself1, Wneigh1, b1,
           Wself2, Wneigh2, b2):
    ei = edge_index.astype(jnp.int32)
    src2d = ei[0].reshape(NWORK, NCHUNK, CH)
    dst2d = ei[1].reshape(NWORK, NCHUNK, CH)
    edges4 = jnp.stack([src2d, dst2d], axis=2)

    wn2p = jnp.zeros((H, H), jnp.float32).at[:C].set(Wneigh2)
    ws2p = jnp.zeros((H, H), jnp.float32).at[:C].set(Wself2)
    b2p = jnp.zeros((1, H), jnp.float32).at[0, :C].set(b2)
    b0r = b0.reshape(1, H)
    b1r = b1.reshape(1, H)

    def _xla_agg(y):
        a = jax.ops.segment_sum(y[ei[0]], ei[1], num_segments=N)
        return jnp.stack([a, jnp.zeros_like(a)])

    degp = _sc_deg(dst2d).T
    y0, s0 = _mm0(features, Wneigh0, Wself0)
    agg0 = _sc_agg_sync(y0, src2d, dst2d)
    y1, s1 = _mid(s0, agg0, degp, b0r, Wneigh1, Wself1)
    agg1 = _sc_agg_sync(y1, src2d, dst2d)
    y2, s2 = _mid(s1, agg1, degp, b1r, wn2p, ws2p)
    agg2 = _sc_agg_sync(y2, src2d, dst2d)
    out = _tail(s2, agg2, degp, b2p)
    return out[:, :C]


# speedup vs baseline: 10.8873x; 1.5806x over previous
import jax
import jax.numpy as jnp
from jax import lax
from jax.experimental import pallas as pl
from jax.experimental.pallas import tpu as pltpu
from jax.experimental.pallas import tpu_sc as plsc

N = 10000
E = 320000
D = 128
H = 128
C = 47

NC = 2
NS = 16
NWORK = NC * NS
EPW = E // NWORK
CH = 80
NCHUNK = EPW // CH
PUB = 1000
NPUB = N // PUB
ZR = 40
NZ = PUB // ZR


def _sc_agg_body(y_hbm, src_hbm, dst_hbm, out_hbm, srcv, dstv, rows, semg,
                 agg_sh):
    cid = lax.axis_index("c")
    sid = lax.axis_index("s")
    wid = cid * NS + sid

    def _zero_row(i, _):
        for j in range(H // 16):
            rows[0, i, pl.ds(j * 16, 16)] = jnp.zeros((16,), jnp.float32)
        return 0
    lax.fori_loop(0, ZR, _zero_row, 0)
    r0 = sid * PUB

    @pl.when(sid < NPUB)
    def _():
        for k in range(NZ):
            pltpu.sync_copy(rows.at[0, pl.ds(0, ZR), :],
                            agg_sh.at[pl.ds(r0 + k * ZR, ZR), :])
    plsc.subcore_barrier()

    off = pl.multiple_of(wid * EPW, 8)
    pltpu.sync_copy(src_hbm.at[pl.ds(off, EPW)], srcv)
    pltpu.sync_copy(dst_hbm.at[wid], dstv)

    def _start(b, c):
        o = pl.multiple_of(c * CH, 8)
        pltpu.async_copy(y_hbm.at[srcv.at[pl.ds(o, CH)]], rows.at[b],
                         semg.at[b])

    def _wait(b):
        pltpu.make_async_copy(y_hbm.at[srcv.at[pl.ds(0, CH)]], rows.at[b],
                              semg.at[b]).wait()

    def _scat(b, c):
        pltpu.sync_copy(rows.at[b], agg_sh.at[dstv.at[c]], add=True)

    _start(0, 0)
    _start(1, 1)

    def _group(g, _):
        c0 = 2 * g
        _wait(0)
        _scat(0, c0)

        @pl.when(c0 + 2 < NCHUNK)
        def _():
            _start(0, c0 + 2)
        _wait(1)
        _scat(1, c0 + 1)

        @pl.when(c0 + 3 < NCHUNK)
        def _():
            _start(1, c0 + 3)
        return 0
    lax.fori_loop(0, (NCHUNK - 1) // 2, _group, 0)
    _wait(0)
    _scat(0, NCHUNK - 1)
    plsc.subcore_barrier()

    @pl.when(sid < NPUB)
    def _():
        sl = pl.ds(r0, PUB)
        pltpu.sync_copy(agg_sh.at[sl, :], out_hbm.at[cid, sl, :])


_sc_agg = pl.kernel(
    _sc_agg_body,
    out_type=jax.ShapeDtypeStruct((NC, N, H), jnp.float32),
    mesh=plsc.VectorSubcoreMesh(core_axis_name="c", subcore_axis_name="s"),
    scratch_types=[
        pltpu.VMEM((EPW,), jnp.int32),
        pltpu.VMEM((NCHUNK, CH), jnp.int32),
        pltpu.VMEM((2, CH, H), jnp.float32),
        pltpu.SemaphoreType.DMA((2,)),
        pltpu.VMEM_SHARED((N, H), jnp.float32),
    ],
)


def _sc_deg_body(dst_hbm, deg_hbm, dstv, hist):
    cid = lax.axis_index("c")
    sid = lax.axis_index("s")
    wid = cid * NS + sid

    def _zero(i, _):
        hist[pl.ds(i * 16, 16)] = jnp.zeros((16,), jnp.float32)
        return 0
    lax.fori_loop(0, N // 16, _zero, 0)

    pltpu.sync_copy(dst_hbm.at[wid], dstv)
    ones = jnp.ones((16,), jnp.float32)

    def _edges(i, _):
        def _vec(k, _):
            idx = dstv[i, pl.ds(k * 16, 16)]
            plsc.addupdate_scatter(hist, [idx], ones)
            return 0
        lax.fori_loop(0, CH // 16, _vec, 0)
        return 0
    lax.fori_loop(0, NCHUNK, _edges, 0)

    pltpu.sync_copy(hist, deg_hbm.at[wid])


_sc_deg = pl.kernel(
    _sc_deg_body,
    out_type=jax.ShapeDtypeStruct((NWORK, N), jnp.float32),
    mesh=plsc.VectorSubcoreMesh(core_axis_name="c", subcore_axis_name="s"),
    scratch_types=[
        pltpu.VMEM((NCHUNK, CH), jnp.int32),
        pltpu.VMEM((N,), jnp.float32),
    ],
    compiler_params=pltpu.CompilerParams(needs_layout_passes=False),
)


_BLK = 1000
_GRID = N // _BLK


def _mm0_body(x_ref, wn_ref, ws_ref, y_ref, s_ref):
    x = x_ref[...]
    y_ref[...] = jnp.dot(x, wn_ref[...].T, preferred_element_type=jnp.float32)
    s_ref[...] = jnp.dot(x, ws_ref[...].T, preferred_element_type=jnp.float32)


def _mm0(x, wn, ws):
    return pl.pallas_call(
        _mm0_body,
        grid=(_GRID,),
        in_specs=[
            pl.BlockSpec((_BLK, D), lambda i: (i, 0)),
            pl.BlockSpec((H, D), lambda i: (0, 0)),
            pl.BlockSpec((H, D), lambda i: (0, 0)),
        ],
        out_specs=[
            pl.BlockSpec((_BLK, H), lambda i: (i, 0)),
            pl.BlockSpec((_BLK, H), lambda i: (i, 0)),
        ],
        out_shape=[jax.ShapeDtypeStruct((N, H), jnp.float32),
                   jax.ShapeDtypeStruct((N, H), jnp.float32)],
    )(x, wn, ws)


def _mid_body(s_ref, agg_ref, deg_ref, b_ref, wn_ref, ws_ref, y_ref, sn_ref):
    agg = agg_ref[0] + agg_ref[1]
    deg = jnp.sum(deg_ref[...], axis=1, keepdims=True)
    inv = 1.0 / jnp.maximum(deg, 1.0)
    h = jnp.maximum(s_ref[...] + agg * inv + b_ref[...], 0.0)
    y_ref[...] = jnp.dot(h, wn_ref[...].T, preferred_element_type=jnp.float32)
    sn_ref[...] = jnp.dot(h, ws_ref[...].T, preferred_element_type=jnp.float32)


def _mid(s, aggp, degp, b, wn, ws):
    return pl.pallas_call(
        _mid_body,
        grid=(_GRID,),
        in_specs=[
            pl.BlockSpec((_BLK, H), lambda i: (i, 0)),
            pl.BlockSpec((NC, _BLK, H), lambda i: (0, i, 0)),
            pl.BlockSpec((_BLK, NWORK), lambda i: (i, 0)),
            pl.BlockSpec((1, H), lambda i: (0, 0)),
            pl.BlockSpec((H, H), lambda i: (0, 0)),
            pl.BlockSpec((H, H), lambda i: (0, 0)),
        ],
        out_specs=[
            pl.BlockSpec((_BLK, H), lambda i: (i, 0)),
            pl.BlockSpec((_BLK, H), lambda i: (i, 0)),
        ],
        out_shape=[jax.ShapeDtypeStruct((N, H), jnp.float32),
                   jax.ShapeDtypeStruct((N, H), jnp.float32)],
    )(s, aggp, degp, b, wn, ws)


def _tail_body(s_ref, agg_ref, deg_ref, b_ref, o_ref):
    agg = agg_ref[0] + agg_ref[1]
    deg = jnp.sum(deg_ref[...], axis=1, keepdims=True)
    inv = 1.0 / jnp.maximum(deg, 1.0)
    o_ref[...] = s_ref[...] + agg * inv + b_ref[...]


def _tail(s, aggp, degp, b):
    return pl.pallas_call(
        _tail_body,
        grid=(_GRID,),
        in_specs=[
            pl.BlockSpec((_BLK, H), lambda i: (i, 0)),
            pl.BlockSpec((NC, _BLK, H), lambda i: (0, i, 0)),
            pl.BlockSpec((_BLK, NWORK), lambda i: (i, 0)),
            pl.BlockSpec((1, H), lambda i: (0, 0)),
        ],
        out_specs=pl.BlockSpec((_BLK, H), lambda i: (i, 0)),
        out_shape=jax.ShapeDtypeStruct((N, H), jnp.float32),
    )(s, aggp, degp, b)


def kernel(features, edge_index, Wself0, Wneigh0, b0, Wself1, Wneigh1, b1,
           Wself2, Wneigh2, b2):
    ei = edge_index.astype(jnp.int32)
    src1d = ei[0]
    dst2d = ei[1].reshape(NWORK, NCHUNK, CH)

    wn2p = jnp.zeros((H, H), jnp.float32).at[:C].set(Wneigh2)
    ws2p = jnp.zeros((H, H), jnp.float32).at[:C].set(Wself2)
    b2p = jnp.zeros((1, H), jnp.float32).at[0, :C].set(b2)
    b0r = b0.reshape(1, H)
    b1r = b1.reshape(1, H)

    degp = _sc_deg(dst2d).T
    y0, s0 = _mm0(features, Wneigh0, Wself0)
    agg0 = _sc_agg(y0, src1d, dst2d)
    y1, s1 = _mid(s0, agg0, degp, b0r, Wneigh1, Wself1)
    agg1 = _sc_agg(y1, src1d, dst2d)
    y2, s2 = _mid(s1, agg1, degp, b1r, wn2p, ws2p)
    agg2 = _sc_agg(y2, src1d, dst2d)
    out = _tail(s2, agg2, degp, b2p)
    return out[:, :C]
